# baseline (speedup 1.0000x reference)
import functools

import jax
import jax.numpy as jnp
from jax import lax
from jax.experimental import pallas as pl
from jax.experimental.pallas import tpu as pltpu
from jax.experimental.pallas import tpu_sc as plsc

B = 16384
D = 32
V = 100001
VPAD = 100096
NUM_CORES = 2
NUM_SUBCORES = 16
NW = NUM_CORES * NUM_SUBCORES
N_TILES = VPAD // 128
SLAB_TILES = 25
SLAB_COLS = SLAB_TILES * 128
N_BANDS = D // 8
IDX_BLK = 4096
POS_SHIFT = 14


def _make_kernel():
    mesh = plsc.VectorSubcoreMesh(core_axis_name="c", subcore_axis_name="s")

    @functools.partial(
        pl.kernel,
        mesh=mesh,
        out_type=jax.ShapeDtypeStruct((B * D,), jnp.float32),
        scratch_types=[
            pltpu.VMEM((2 * IDX_BLK,), jnp.int32),
            pltpu.VMEM((N_BANDS, 8, SLAB_COLS), jnp.float32),
            pltpu.VMEM((B + 16,), jnp.int32),
            pltpu.VMEM((16, D), jnp.float32),
            pltpu.VMEM((16 * D,), jnp.float32),
            pltpu.SemaphoreType.DMA,
            pltpu.SemaphoreType.DMA,
            pltpu.SemaphoreType.DMA,
        ],
        compiler_params=pltpu.CompilerParams(
            use_tc_tiling_on_sc=True,
            needs_layout_passes=False,
            disable_bounds_checks=True,
            disable_semaphore_checks=True,
        ),
    )
    def k(
        tbl_hbm, idx_hbm, out_hbm, idx_v, slab_v, hits_v, ring_v, drain_v,
        sem, osem, isem,
    ):
        wid = lax.axis_index("s") * NUM_CORES + lax.axis_index("c")
        small = jnp.int32(25 * 14)
        t0 = jnp.where(wid < 14, 25 * wid, small + 24 * (wid - 14))
        ntc = jnp.where(wid < 14, 25, 24)
        slab_t0 = jnp.minimum(t0, N_TILES - SLAB_TILES)
        slab_c0 = slab_t0 * 128
        sel_a = t0 * 128
        sel_b = (t0 + ntc) * 128

        slab_cps = []
        for band in range(N_BANDS):
            slab_cps.append(
                pltpu.async_copy(
                    tbl_hbm.at[pl.ds(band * 8, 8), pl.ds(slab_c0, SLAB_COLS)],
                    slab_v.at[band],
                    sem,
                )
            )

        lane = lax.iota(jnp.int32, 16)
        zeros = jnp.full((16,), 0, jnp.int32)
        sel_a_v = zeros + sel_a
        sel_b_v = zeros + sel_b
        c0_v = zeros + slab_c0

        total = jnp.int32(0)
        idx_cp = pltpu.async_copy(
            idx_hbm.at[pl.ds(0, IDX_BLK)], idx_v.at[pl.ds(0, IDX_BLK)], isem
        )
        for blk in range(B // IDX_BLK):
            idx_cp.wait()
            if blk + 1 < B // IDX_BLK:
                idx_cp = pltpu.async_copy(
                    idx_hbm.at[pl.ds((blk + 1) * IDX_BLK, IDX_BLK)],
                    idx_v.at[pl.ds(((blk + 1) % 2) * IDX_BLK, IDX_BLK)],
                    isem,
                )
            ibase = (blk % 2) * IDX_BLK

            def sel_body(g, off, blk=blk, ibase=ibase):
                vec = idx_v[pl.ds(ibase + g * 16, 16)]
                m = jnp.logical_and(vec >= sel_a_v, vec < sel_b_v)
                cnt = plsc.all_reduce_population_count(m)
                pos_v = lane + (blk * IDX_BLK + g * 16)
                packed = pos_v + lax.shift_left(vec - c0_v, POS_SHIFT)
                plsc.store_compressed(hits_v.at[pl.ds(off, 16)], packed, mask=m)
                return off + cnt[0]

            total = lax.fori_loop(0, IDX_BLK // 16, sel_body, total)

        first_vec = hits_v[pl.ds(0, 16)]
        first = zeros + first_vec[0]

        @pl.when(total > 0)
        def _():
            hits_v[pl.ds(total, 16)] = first

        for c in slab_cps:
            c.wait()

        band_idx, sub_idx = [], []
        for h in range(2):
            d = lane + h * 16
            band_idx.append(lax.shift_right_logical(d, 3))
            sub_idx.append(d & 7)
        pos_mask = zeros + ((1 << POS_SHIFT) - 1)
        n_grp = lax.shift_right_logical(total + 15, 4)

        def ext_body(eg, carry):
            pk = hits_v[pl.ds(eg * 16, 16)]
            pos_v = pk & pos_mask
            col_v = lax.shift_right_logical(pk, POS_SHIFT)
            for e in range(16):
                col = zeros + col_v[e]
                for h in range(2):
                    ring_v[e, pl.ds(h * 16, 16)] = plsc.load_gather(
                        slab_v, [band_idx[h], sub_idx[h], col]
                    )
                pltpu.async_copy(
                    ring_v.at[e],
                    out_hbm.at[pl.ds(pos_v[e] * D, D)],
                    osem,
                )
            pltpu.make_async_copy(
                out_hbm.at[pl.ds(0, 16 * D)], drain_v, osem
            ).wait()
            return carry

        lax.fori_loop(0, n_grp, ext_body, jnp.int32(0))

    return k


@jax.jit
def kernel(template_table, eventids):
    idx = eventids.astype(jnp.int32)
    tbl_t = template_table.T
    out1d = _make_kernel()(tbl_t, idx)
    return out1d.reshape(B, D)
